# Initial kernel scaffold; baseline (speedup 1.0000x reference)
import jax
import jax.numpy as jnp
from jax.experimental import pallas as pl


def kernel(atomic_embedding, partial_charges, pair_indices, gs, gv, agh, W1, b1, W2, b2, W3, b3):
    raise NotImplementedError("write your pallas kernel here")


# speedup vs baseline: 4.1553x; 4.1553x over previous
import functools

import jax
import jax.numpy as jnp
import numpy as np
from jax import lax
from jax.experimental import pallas as pl
from jax.experimental.pallas import tpu as pltpu
from jax.experimental.pallas import tpu_sc as plsc

N = 10000
P = 160000
F = 128
G = 16
H = 8
HID = 128
OUT_FEAT = F + 2

NPAD = 10240
NW = 32
CHUNK = 128
PPAD = 163840
PW = PPAD // NW
CHN = PW // CHUNK
NACC = 10240
AW = 9
ACC = NACC * AW

_BM0 = 1024
_BP = 2048
_BM4 = 1000


def _mm_body(e_ref, w_ref, o_ref):
    o_ref[...] = jnp.dot(e_ref[...], w_ref[...],
                         preferred_element_type=jnp.float32)


def _pair_body(tj_ref, gv_ref, gs_ref, idx_ref, rep_ref, coll_ref, sel8_ref,
               sel1_ref, o_ref, a_ref):
    tj = tj_ref[...]
    vs = jnp.zeros((_BP, H), jnp.float32)
    for d in range(3):
        gvd = gv_ref[:, d * G:(d + 1) * G]
        gvr = jnp.dot(gvd, rep_ref[...], preferred_element_type=jnp.float32)
        cd = jnp.dot(gvr * tj, coll_ref[...],
                     preferred_element_type=jnp.float32)
        vs = vs + cd * cd
    s = jnp.sum(gs_ref[...], axis=1, keepdims=True)
    o_ref[...] = (jnp.dot(vs, sel8_ref[...],
                          preferred_element_type=jnp.float32)
                  + s * sel1_ref[...])
    lanes = lax.broadcasted_iota(jnp.int32, (_BP, 16), 1)
    a_ref[...] = idx_ref[...] * AW + lanes


def _reduce_body(p_ref, o_ref):
    o_ref[...] = jnp.sum(p_ref[...], axis=0, keepdims=True)


def _mlp_body(acc_ref, e_ref, q_ref, w1a_ref, w1b_ref, w1c_ref,
              b1_ref, w2_ref, b2_ref, w3_ref, b3_ref, o_ref):
    acc = acc_ref[...]
    vs = acc[:, 0:H]
    s = acc[:, H:H + 1]
    e = e_ref[...]
    q = q_ref[...]
    h = (jnp.dot(e * s, w1a_ref[...], preferred_element_type=jnp.float32)
         + jnp.dot(vs, w1b_ref[...], preferred_element_type=jnp.float32)
         + (q * s) * w1c_ref[...]
         + b1_ref[...])
    h = jax.nn.gelu(h)
    h = jax.nn.gelu(jnp.dot(h, w2_ref[...],
                            preferred_element_type=jnp.float32) + b2_ref[...])
    o_ref[...] = (jnp.dot(h, w3_ref[...], preferred_element_type=jnp.float32)
                  + b3_ref[...])


@functools.lru_cache(maxsize=None)
def _sc_kernels():
    mesh = plsc.VectorSubcoreMesh(core_axis_name="c", subcore_axis_name="s")

    @functools.partial(
        pl.kernel,
        out_type=jax.ShapeDtypeStruct((PPAD, F), jnp.float32),
        mesh=mesh,
        scratch_types=[
            pltpu.VMEM((CHN, CHUNK), jnp.int32),
            pltpu.VMEM((CHUNK, F), jnp.float32),
            pltpu.SemaphoreType.DMA,
        ],
    )
    def gather_k(t_hbm, idx_hbm, out_hbm, idx_v, rows_v, sem):
        c = lax.axis_index("c")
        s = lax.axis_index("s")
        wid = s * 2 + c
        pltpu.sync_copy(idx_hbm.at[wid], idx_v)

        def body(j, carry):
            pltpu.async_copy(t_hbm.at[idx_v.at[j]], rows_v, sem).wait()
            pltpu.sync_copy(rows_v,
                            out_hbm.at[pl.ds(wid * PW + j * CHUNK, CHUNK)])
            return carry

        lax.fori_loop(0, CHN, body, 0)

    @functools.partial(
        pl.kernel,
        out_type=jax.ShapeDtypeStruct((NW, ACC), jnp.float32),
        mesh=mesh,
        compiler_params=pltpu.CompilerParams(needs_layout_passes=False),
        scratch_types=[
            pltpu.VMEM((CHUNK, 16), jnp.int32),
            pltpu.VMEM((CHUNK, 16), jnp.float32),
            pltpu.VMEM((ACC,), jnp.float32),
        ],
    )
    def scatter_k(vals_hbm, addr_hbm, out_hbm, addr_v, vals_v, acc):
        c = lax.axis_index("c")
        s = lax.axis_index("s")
        wid = s * 2 + c
        zeros16 = jnp.zeros((16,), jnp.float32)
        lane_ok = lax.iota(jnp.int32, 16) < AW

        def zbody(i, carry):
            acc[pl.ds(i * 16, 16)] = zeros16
            return carry

        lax.fori_loop(0, ACC // 16, zbody, 0)

        def chunk_body(j, carry):
            pltpu.sync_copy(vals_hbm.at[pl.ds(wid * PW + j * CHUNK, CHUNK)],
                            vals_v)
            pltpu.sync_copy(addr_hbm.at[pl.ds(wid * PW + j * CHUNK, CHUNK)],
                            addr_v)

            def pair_body(p, carry2):
                plsc.addupdate_scatter(acc, [addr_v[p]], vals_v[p],
                                       mask=lane_ok)
                return carry2

            lax.fori_loop(0, CHUNK, pair_body, 0, unroll=8)
            return carry

        lax.fori_loop(0, CHN, chunk_body, 0)
        pltpu.sync_copy(acc, out_hbm.at[wid])

    return gather_k, scatter_k


def kernel(atomic_embedding, partial_charges, pair_indices, gs, gv, agh,
           W1, b1, W2, b2, W3, b3):
    idx_j = pair_indices[1]
    e_pad = jnp.pad(atomic_embedding, ((0, NPAD - N), (0, 0)))
    agh_flat = agh.reshape(F, G * H)
    idx_pad = jnp.pad(idx_j, (0, PPAD - P), constant_values=N)
    idx_w = idx_pad.reshape(NW, CHN, CHUNK)
    gs_pad = jnp.pad(gs, ((0, PPAD - P), (0, 0)))
    gv_pad = jnp.pad(gv.reshape(P, 3 * G), ((0, PPAD - P), (0, 0)))

    rep = jnp.asarray(np.repeat(np.eye(G, dtype=np.float32), H, axis=1))
    coll = jnp.asarray(np.tile(np.eye(H, dtype=np.float32), (G, 1)))
    sel8 = jnp.asarray(np.eye(H, 16, dtype=np.float32))
    sel1 = jnp.asarray(np.eye(1, 16, k=H, dtype=np.float32))

    t = pl.pallas_call(
        _mm_body,
        grid=(NPAD // _BM0,),
        in_specs=[
            pl.BlockSpec((_BM0, F), lambda i: (i, 0)),
            pl.BlockSpec((F, G * H), lambda i: (0, 0)),
        ],
        out_specs=pl.BlockSpec((_BM0, G * H), lambda i: (i, 0)),
        out_shape=jax.ShapeDtypeStruct((NPAD, G * H), jnp.float32),
    )(e_pad, agh_flat)

    gather_k, scatter_k = _sc_kernels()
    t_j = gather_k(t, idx_w)

    vals, addrs = pl.pallas_call(
        _pair_body,
        grid=(PPAD // _BP,),
        in_specs=[
            pl.BlockSpec((_BP, F), lambda i: (i, 0)),
            pl.BlockSpec((_BP, 3 * G), lambda i: (i, 0)),
            pl.BlockSpec((_BP, G), lambda i: (i, 0)),
            pl.BlockSpec((_BP, 1), lambda i: (i, 0)),
            pl.BlockSpec((G, G * H), lambda i: (0, 0)),
            pl.BlockSpec((G * H, H), lambda i: (0, 0)),
            pl.BlockSpec((H, 16), lambda i: (0, 0)),
            pl.BlockSpec((1, 16), lambda i: (0, 0)),
        ],
        out_specs=[
            pl.BlockSpec((_BP, 16), lambda i: (i, 0)),
            pl.BlockSpec((_BP, 16), lambda i: (i, 0)),
        ],
        out_shape=[
            jax.ShapeDtypeStruct((PPAD, 16), jnp.float32),
            jax.ShapeDtypeStruct((PPAD, 16), jnp.int32),
        ],
    )(t_j, gv_pad, gs_pad, idx_pad.reshape(PPAD, 1), rep, coll, sel8, sel1)

    partials = scatter_k(vals, addrs)

    _BF = ACC // 8
    acc_flat = pl.pallas_call(
        _reduce_body,
        grid=(8,),
        in_specs=[pl.BlockSpec((NW, _BF), lambda i: (0, i))],
        out_specs=pl.BlockSpec((1, _BF), lambda i: (0, i)),
        out_shape=jax.ShapeDtypeStruct((1, ACC), jnp.float32),
    )(partials)
    acc2d = acc_flat.reshape(NACC, AW)

    out = pl.pallas_call(
        _mlp_body,
        grid=(N // _BM4,),
        in_specs=[
            pl.BlockSpec((_BM4, AW), lambda i: (i, 0)),
            pl.BlockSpec((_BM4, F), lambda i: (i, 0)),
            pl.BlockSpec((_BM4, 1), lambda i: (i, 0)),
            pl.BlockSpec((F, HID), lambda i: (0, 0)),
            pl.BlockSpec((H, HID), lambda i: (0, 0)),
            pl.BlockSpec((1, HID), lambda i: (0, 0)),
            pl.BlockSpec((1, HID), lambda i: (0, 0)),
            pl.BlockSpec((HID, HID), lambda i: (0, 0)),
            pl.BlockSpec((1, HID), lambda i: (0, 0)),
            pl.BlockSpec((HID, OUT_FEAT), lambda i: (0, 0)),
            pl.BlockSpec((1, OUT_FEAT), lambda i: (0, 0)),
        ],
        out_specs=pl.BlockSpec((_BM4, OUT_FEAT), lambda i: (i, 0)),
        out_shape=jax.ShapeDtypeStruct((N, OUT_FEAT), jnp.float32),
    )(acc2d, atomic_embedding, partial_charges,
      W1[0:F], W1[F:F + H], W1[F + H:F + H + 1].reshape(1, HID),
      b1.reshape(1, HID), W2, b2.reshape(1, HID), W3,
      b3.reshape(1, OUT_FEAT))

    delta_q = out[:, 0:1]
    f = out[:, 1:2]
    delta_a = out[:, 2:]
    return (delta_a, delta_q, f)


# speedup vs baseline: 4.3302x; 1.0421x over previous
import functools

import jax
import jax.numpy as jnp
import numpy as np
from jax import lax
from jax.experimental import pallas as pl
from jax.experimental.pallas import tpu as pltpu
from jax.experimental.pallas import tpu_sc as plsc

N = 10000
P = 160000
F = 128
G = 16
H = 8
HID = 128
OUT_FEAT = F + 2

NPAD = 10240
NW = 32
CHUNK = 64
PPAD = 163840
PW = PPAD // NW
CHN = PW // CHUNK
SW = P // NW
SCHUNK = 200
SCHN = SW // SCHUNK
NACC = 10016
AW = 9
ACC = NACC * AW

_BM0 = 1024
_BP = 2000
_BM4 = 1000


def _mm_body(e_ref, w_ref, o_ref):
    o_ref[...] = jnp.dot(e_ref[...], w_ref[...],
                         preferred_element_type=jnp.float32)


def _pair_body(tj_ref, gv_ref, gs_ref, idx_ref, rep_ref, coll_ref, sel8_ref,
               sel1_ref, o_ref, a_ref):
    tj = tj_ref[...]
    vs = jnp.zeros((_BP, H), jnp.float32)
    for d in range(3):
        gvd = gv_ref[:, d * G:(d + 1) * G]
        gvr = jnp.dot(gvd, rep_ref[...], preferred_element_type=jnp.float32)
        cd = jnp.dot(gvr * tj, coll_ref[...],
                     preferred_element_type=jnp.float32)
        vs = vs + cd * cd
    s = jnp.sum(gs_ref[...], axis=1, keepdims=True)
    o_ref[...] = (jnp.dot(vs, sel8_ref[...],
                          preferred_element_type=jnp.float32)
                  + s * sel1_ref[...])
    lanes = lax.broadcasted_iota(jnp.int32, (_BP, 16), 1)
    a_ref[...] = idx_ref[...] * AW + lanes


def _reduce_body(p_ref, o_ref):
    o_ref[...] = jnp.sum(p_ref[...], axis=0, keepdims=True)


def _mlp_body(acc_ref, e_ref, q_ref, w1a_ref, w1b_ref, w1c_ref,
              b1_ref, w2_ref, b2_ref, w3_ref, b3_ref, o_ref):
    acc = acc_ref[...]
    vs = acc[:, 0:H]
    s = acc[:, H:H + 1]
    e = e_ref[...]
    q = q_ref[...]
    h = (jnp.dot(e * s, w1a_ref[...], preferred_element_type=jnp.float32)
         + jnp.dot(vs, w1b_ref[...], preferred_element_type=jnp.float32)
         + (q * s) * w1c_ref[...]
         + b1_ref[...])
    h = jax.nn.gelu(h)
    h = jax.nn.gelu(jnp.dot(h, w2_ref[...],
                            preferred_element_type=jnp.float32) + b2_ref[...])
    o_ref[...] = (jnp.dot(h, w3_ref[...], preferred_element_type=jnp.float32)
                  + b3_ref[...])


@functools.lru_cache(maxsize=None)
def _sc_kernels():
    mesh = plsc.VectorSubcoreMesh(core_axis_name="c", subcore_axis_name="s")

    @functools.partial(
        pl.kernel,
        out_type=jax.ShapeDtypeStruct((PPAD, F), jnp.float32),
        mesh=mesh,
        scratch_types=[
            pltpu.VMEM((CHN, CHUNK), jnp.int32),
            pltpu.VMEM((CHUNK, F), jnp.float32),
            pltpu.VMEM((CHUNK, F), jnp.float32),
            pltpu.SemaphoreType.DMA,
            pltpu.SemaphoreType.DMA,
            pltpu.SemaphoreType.DMA,
            pltpu.SemaphoreType.DMA,
        ],
    )
    def gather_k(t_hbm, idx_hbm, out_hbm, idx_v, rows0, rows1,
                 gsem0, gsem1, osem0, osem1):
        c = lax.axis_index("c")
        s = lax.axis_index("s")
        wid = s * 2 + c
        pltpu.sync_copy(idx_hbm.at[wid], idx_v)
        bufs = (rows0, rows1)
        gsems = (gsem0, gsem1)
        osems = (osem0, osem1)

        def gstart(j):
            return pltpu.async_copy(t_hbm.at[idx_v.at[j]], bufs[j % 2],
                                    gsems[j % 2])

        gd = {0: gstart(0)}
        od = {}
        for j in range(CHN):
            gd[j].wait()
            od[j] = pltpu.async_copy(
                bufs[j % 2], out_hbm.at[pl.ds(wid * PW + j * CHUNK, CHUNK)],
                osems[j % 2])
            if j + 1 < CHN:
                if j >= 1:
                    od[j - 1].wait()
                gd[j + 1] = gstart(j + 1)
        od[CHN - 2].wait()
        od[CHN - 1].wait()

    @functools.partial(
        pl.kernel,
        out_type=jax.ShapeDtypeStruct((NW, ACC), jnp.float32),
        mesh=mesh,
        compiler_params=pltpu.CompilerParams(needs_layout_passes=False),
        scratch_types=[
            pltpu.VMEM((SCHUNK * 16,), jnp.int32),
            pltpu.VMEM((SCHUNK * 16,), jnp.int32),
            pltpu.VMEM((SCHUNK * 16,), jnp.float32),
            pltpu.VMEM((SCHUNK * 16,), jnp.float32),
            pltpu.VMEM((ACC,), jnp.float32),
            pltpu.SemaphoreType.DMA,
            pltpu.SemaphoreType.DMA,
        ],
    )
    def scatter_k(vals_hbm, addr_hbm, out_hbm, addr0, addr1, vals0, vals1,
                  acc, sem0, sem1):
        c = lax.axis_index("c")
        s = lax.axis_index("s")
        wid = s * 2 + c
        zeros16 = jnp.zeros((16,), jnp.float32)
        lane_ok = lax.iota(jnp.int32, 16) < AW
        abufs = (addr0, addr1)
        vbufs = (vals0, vals1)
        sems = (sem0, sem1)

        def dstart(j):
            sl = pl.ds((wid * SW + j * SCHUNK) * 16, SCHUNK * 16)
            d0 = pltpu.async_copy(vals_hbm.at[sl], vbufs[j % 2], sems[j % 2])
            d1 = pltpu.async_copy(addr_hbm.at[sl], abufs[j % 2], sems[j % 2])
            return (d0, d1)

        ds = {0: dstart(0)}

        def zbody(i, carry):
            acc[pl.ds(i * 16, 16)] = zeros16
            return carry

        lax.fori_loop(0, ACC // 16, zbody, 0)

        for j in range(SCHN):
            for d in ds.pop(j):
                d.wait()
            if j + 1 < SCHN:
                ds[j + 1] = dstart(j + 1)
            av = abufs[j % 2]
            vv = vbufs[j % 2]

            def pair_body(p, carry2):
                plsc.addupdate_scatter(acc, [av[pl.ds(p * 16, 16)]],
                                       vv[pl.ds(p * 16, 16)], mask=lane_ok)
                return carry2

            lax.fori_loop(0, SCHUNK, pair_body, 0, unroll=10)
        pltpu.sync_copy(acc, out_hbm.at[wid])

    return gather_k, scatter_k


def kernel(atomic_embedding, partial_charges, pair_indices, gs, gv, agh,
           W1, b1, W2, b2, W3, b3):
    idx_j = pair_indices[1]
    e_pad = jnp.pad(atomic_embedding, ((0, NPAD - N), (0, 0)))
    agh_flat = agh.reshape(F, G * H)
    idx_pad = jnp.pad(idx_j, (0, PPAD - P), constant_values=N)
    idx_w = idx_pad.reshape(NW, CHN, CHUNK)
    gv2 = gv.reshape(P, 3 * G)

    rep = jnp.asarray(np.repeat(np.eye(G, dtype=np.float32), H, axis=1))
    coll = jnp.asarray(np.tile(np.eye(H, dtype=np.float32), (G, 1)))
    sel8 = jnp.asarray(np.eye(H, 16, dtype=np.float32))
    sel1 = jnp.asarray(np.eye(1, 16, k=H, dtype=np.float32))

    t = pl.pallas_call(
        _mm_body,
        grid=(NPAD // _BM0,),
        in_specs=[
            pl.BlockSpec((_BM0, F), lambda i: (i, 0)),
            pl.BlockSpec((F, G * H), lambda i: (0, 0)),
        ],
        out_specs=pl.BlockSpec((_BM0, G * H), lambda i: (i, 0)),
        out_shape=jax.ShapeDtypeStruct((NPAD, G * H), jnp.float32),
    )(e_pad, agh_flat)

    gather_k, scatter_k = _sc_kernels()
    t_j = gather_k(t, idx_w)

    vals, addrs = pl.pallas_call(
        _pair_body,
        grid=(P // _BP,),
        in_specs=[
            pl.BlockSpec((_BP, F), lambda i: (i, 0)),
            pl.BlockSpec((_BP, 3 * G), lambda i: (i, 0)),
            pl.BlockSpec((_BP, G), lambda i: (i, 0)),
            pl.BlockSpec((_BP, 1), lambda i: (i, 0)),
            pl.BlockSpec((G, G * H), lambda i: (0, 0)),
            pl.BlockSpec((G * H, H), lambda i: (0, 0)),
            pl.BlockSpec((H, 16), lambda i: (0, 0)),
            pl.BlockSpec((1, 16), lambda i: (0, 0)),
        ],
        out_specs=[
            pl.BlockSpec((_BP, 16), lambda i: (i, 0)),
            pl.BlockSpec((_BP, 16), lambda i: (i, 0)),
        ],
        out_shape=[
            jax.ShapeDtypeStruct((P, 16), jnp.float32),
            jax.ShapeDtypeStruct((P, 16), jnp.int32),
        ],
    )(t_j, gv2, gs, idx_j.reshape(P, 1), rep, coll, sel8, sel1)

    partials = scatter_k(vals.reshape(P * 16), addrs.reshape(P * 16))

    acc_flat = pl.pallas_call(
        _reduce_body,
        grid=(1,),
        in_specs=[pl.BlockSpec((NW, ACC), lambda i: (0, 0))],
        out_specs=pl.BlockSpec((1, ACC), lambda i: (0, 0)),
        out_shape=jax.ShapeDtypeStruct((1, ACC), jnp.float32),
    )(partials)
    acc2d = acc_flat.reshape(NACC, AW)

    out = pl.pallas_call(
        _mlp_body,
        grid=(N // _BM4,),
        in_specs=[
            pl.BlockSpec((_BM4, AW), lambda i: (i, 0)),
            pl.BlockSpec((_BM4, F), lambda i: (i, 0)),
            pl.BlockSpec((_BM4, 1), lambda i: (i, 0)),
            pl.BlockSpec((F, HID), lambda i: (0, 0)),
            pl.BlockSpec((H, HID), lambda i: (0, 0)),
            pl.BlockSpec((1, HID), lambda i: (0, 0)),
            pl.BlockSpec((1, HID), lambda i: (0, 0)),
            pl.BlockSpec((HID, HID), lambda i: (0, 0)),
            pl.BlockSpec((1, HID), lambda i: (0, 0)),
            pl.BlockSpec((HID, OUT_FEAT), lambda i: (0, 0)),
            pl.BlockSpec((1, OUT_FEAT), lambda i: (0, 0)),
        ],
        out_specs=pl.BlockSpec((_BM4, OUT_FEAT), lambda i: (i, 0)),
        out_shape=jax.ShapeDtypeStruct((N, OUT_FEAT), jnp.float32),
    )(acc2d, atomic_embedding, partial_charges,
      W1[0:F], W1[F:F + H], W1[F + H:F + H + 1].reshape(1, HID),
      b1.reshape(1, HID), W2, b2.reshape(1, HID), W3,
      b3.reshape(1, OUT_FEAT))

    delta_q = out[:, 0:1]
    f = out[:, 1:2]
    delta_a = out[:, 2:]
    return (delta_a, delta_q, f)
